# Initial kernel scaffold; baseline (speedup 1.0000x reference)
import jax
import jax.numpy as jnp
from jax.experimental import pallas as pl


def kernel(categorical_tensor, tables):
    raise NotImplementedError("write your pallas kernel here")


# speedup vs baseline: 4.9179x; 4.9179x over previous
import functools

import jax
import jax.numpy as jnp
from jax import lax
from jax.experimental import pallas as pl
from jax.experimental.pallas import tpu as pltpu
from jax.experimental.pallas import tpu_sc as plsc

NUM_TABLES = 26
VOCAB_SZ = 100000
BATCH_SZ = 16384

NUM_CORES = 2
NUM_SUBCORES = 16
LANES = 16

CHUNK = 8192


def _emb_body(tables_hbm, idx_hbm, out_hbm, table_v, idx_v, out_v):
    wid = lax.axis_index("s") * NUM_CORES + lax.axis_index("c")

    @pl.when(wid < NUM_TABLES)
    def _():
        pltpu.sync_copy(tables_hbm.at[wid], table_v)
        for chunk in range(BATCH_SZ // CHUNK):
            off = chunk * CHUNK
            pltpu.sync_copy(idx_hbm.at[wid, pl.ds(off, CHUNK)], idx_v)

            @pl.loop(0, CHUNK // LANES, unroll=8)
            def _gather(i):
                sl = pl.ds(i * LANES, LANES)
                out_v[sl] = plsc.load_gather(table_v, [idx_v[sl]])

            pltpu.sync_copy(out_v, out_hbm.at[wid, pl.ds(off, CHUNK)])


@functools.partial(
    pl.kernel,
    out_type=jax.ShapeDtypeStruct((NUM_TABLES, BATCH_SZ), jnp.float32),
    mesh=plsc.VectorSubcoreMesh(core_axis_name="c", subcore_axis_name="s"),
    scratch_types=[
        pltpu.VMEM((VOCAB_SZ,), jnp.float32),
        pltpu.VMEM((CHUNK,), jnp.int32),
        pltpu.VMEM((CHUNK,), jnp.float32),
    ],
    compiler_params=pltpu.CompilerParams(needs_layout_passes=False),
)
def _emb_kernel(tables_hbm, idx_hbm, out_hbm, table_v, idx_v, out_v):
    _emb_body(tables_hbm, idx_hbm, out_hbm, table_v, idx_v, out_v)


def kernel(categorical_tensor, tables):
    idx_t = categorical_tensor.astype(jnp.int32).T
    tables2 = tables.reshape(NUM_TABLES, VOCAB_SZ)
    out_t = _emb_kernel(tables2, idx_t)
    return out_t.T
